# baseline (speedup 1.0000x reference)
import functools

import jax
import jax.numpy as jnp
from jax import lax
from jax.experimental import pallas as pl
from jax.experimental.pallas import tpu as pltpu
from jax.experimental.pallas import tpu_sc as plsc

_ROWS = 1000000
_D = 32
_B = 16384
_F = 26
_TOTAL = _B * _F
_NC = 2
_NS = 16
_NW = _NC * _NS
_PW = _TOTAL // _NW

_mesh = plsc.VectorSubcoreMesh(core_axis_name="c", subcore_axis_name="s")


@functools.partial(
    pl.kernel,
    out_type=jax.ShapeDtypeStruct((_TOTAL,), jnp.int32),
    mesh=_mesh,
    scratch_types=[pltpu.VMEM((_PW,), jnp.int32)],
    compiler_params=pltpu.CompilerParams(use_tc_tiling_on_sc=True),
)
def _xprep(xt_hbm, out_hbm, buf):
    wid = lax.axis_index("s") * _NC + lax.axis_index("c")
    for w in range(_NW):
        @pl.when(wid == w)
        def _():
            base = w * _PW
            f0, b0 = divmod(base, _B)
            n0 = min(_B - b0, _PW)
            pltpu.sync_copy(xt_hbm.at[f0, pl.ds(b0, n0)], buf.at[pl.ds(0, n0)])
            if n0 < _PW:
                pltpu.sync_copy(
                    xt_hbm.at[f0 + 1, pl.ds(0, _PW - n0)],
                    buf.at[pl.ds(n0, _PW - n0)],
                )
    pltpu.sync_copy(buf, out_hbm.at[pl.ds(wid * _PW, _PW)])


_QB = 512
_NQ = _B // _QB
_UNITS_PW = _F * _NQ // _NW


@functools.partial(
    pl.kernel,
    out_type=jax.ShapeDtypeStruct((_TOTAL * _D,), jnp.float32),
    mesh=_mesh,
    scratch_types=[
        pltpu.VMEM((_QB,), jnp.int32),
        pltpu.VMEM((_QB, _D), jnp.float32),
        pltpu.VMEM((_QB * _D,), jnp.float32),
        pltpu.SemaphoreType.DMA,
    ],
    compiler_params=pltpu.CompilerParams(
        use_tc_tiling_on_sc=False, needs_layout_passes=False
    ),
)
def _gather5(idx_hbm, table_hbm, out_hbm, iv, gbuf, obuf, gsem):
    wid = lax.axis_index("s") * _NC + lax.axis_index("c")
    lanes = jnp.arange(16, dtype=jnp.int32)

    def do_unit(u, _):
        f = u // _NQ
        q = u % _NQ
        pltpu.sync_copy(idx_hbm.at[pl.ds(f * _B + q * _QB, _QB)], iv)
        pltpu.async_copy(table_hbm.at[iv], gbuf, gsem).wait()
        def do_tc8(t, _):
            rbase = lanes + t * 128
            for d in range(_D):
                tr, rr = d // 8, d % 8
                for c0 in range(0, 128, 16):
                    g = plsc.load_gather(
                        gbuf, [rbase + c0, jnp.full((16,), d, jnp.int32)]
                    )
                    obuf[pl.ds(tr * 4096 + t * 1024 + rr * 128 + c0, 16)] = g
            return _
        lax.fori_loop(0, 4, do_tc8, 0)
        for tr in range(4):
            pltpu.sync_copy(
                obuf.at[pl.ds(tr * 4096, 4096)],
                out_hbm.at[pl.ds(((f * 4 + tr) * 128 + q * 4) * 1024, 4096)],
            )
        return _

    lax.fori_loop(wid * _UNITS_PW, (wid + 1) * _UNITS_PW, do_unit, 0)


def kernel(x, table):
    xt = x.T.astype(jnp.int32)
    idx = _xprep(xt)
    o5 = _gather5(idx, table)
    return (
        o5.reshape(_F, 4, 128, 8, 128)
        .transpose(2, 4, 0, 1, 3)
        .reshape(_B, _F, _D)
    )


# speedup vs baseline: 1.3300x; 1.3300x over previous
import functools

import jax
import jax.numpy as jnp
from jax import lax
from jax.experimental import pallas as pl
from jax.experimental.pallas import tpu as pltpu
from jax.experimental.pallas import tpu_sc as plsc

_ROWS = 1000000
_D = 32
_B = 16384
_F = 26
_TOTAL = _B * _F
_NC = 2
_NS = 16
_NW = _NC * _NS
_PW = _TOTAL // _NW

_mesh = plsc.VectorSubcoreMesh(core_axis_name="c", subcore_axis_name="s")


@functools.partial(
    pl.kernel,
    out_type=jax.ShapeDtypeStruct((_TOTAL,), jnp.int32),
    mesh=_mesh,
    scratch_types=[pltpu.VMEM((_PW,), jnp.int32)],
    compiler_params=pltpu.CompilerParams(use_tc_tiling_on_sc=True),
)
def _xprep(xt_hbm, out_hbm, buf):
    wid = lax.axis_index("s") * _NC + lax.axis_index("c")
    for w in range(_NW):
        @pl.when(wid == w)
        def _():
            base = w * _PW
            f0, b0 = divmod(base, _B)
            n0 = min(_B - b0, _PW)
            pltpu.sync_copy(xt_hbm.at[f0, pl.ds(b0, n0)], buf.at[pl.ds(0, n0)])
            if n0 < _PW:
                pltpu.sync_copy(
                    xt_hbm.at[f0 + 1, pl.ds(0, _PW - n0)],
                    buf.at[pl.ds(n0, _PW - n0)],
                )
    pltpu.sync_copy(buf, out_hbm.at[pl.ds(wid * _PW, _PW)])


_QB = 1024
_NQ = _B // _QB
_UNITS_PW = _F * _NQ // _NW
_TRS = _QB * 8


@functools.partial(
    pl.kernel,
    out_type=jax.ShapeDtypeStruct((_TOTAL * _D,), jnp.float32),
    mesh=_mesh,
    scratch_types=[
        pltpu.VMEM((_QB,), jnp.int32),
        pltpu.VMEM((_QB, _D), jnp.float32),
        pltpu.VMEM((_QB * _D,), jnp.float32),
        pltpu.SemaphoreType.DMA,
        pltpu.SemaphoreType.DMA,
    ],
    compiler_params=pltpu.CompilerParams(
        use_tc_tiling_on_sc=False, needs_layout_passes=False
    ),
)
def _gather5(idx_hbm, table_hbm, out_hbm, iv, gbuf, obuf, gsem, osem):
    wid = lax.axis_index("s") * _NC + lax.axis_index("c")
    lanes = jnp.arange(16, dtype=jnp.int32)
    colv = [(lanes + s) % 16 for s in range(16)]
    dd = [(jnp.arange(16) + s) % 16 for s in range(16)]
    q0 = [(d // 8) * _TRS + (d % 8) * 128 + jnp.arange(16) for d in dd]

    def do_unit(u, _):
        f = u // _NQ
        q = u % _NQ
        pltpu.sync_copy(idx_hbm.at[pl.ds(f * _B + q * _QB, _QB)], iv)
        pltpu.async_copy(table_hbm.at[iv], gbuf, gsem).wait()
        def do_tc(t, _):
            tsplat = jnp.full((16,), 0, jnp.int32) + t * 1024
            rbase = lanes + t * 128
            for c0 in range(0, 128, 16):
                rowv = rbase + c0
                for d0 in (0, 16):
                    for s in range(16):
                        g = plsc.load_gather(gbuf, [rowv, colv[s] + d0])
                        pos = (q0[s] + ((d0 // 8) * _TRS + c0)).astype(jnp.int32)
                        plsc.store_scatter(obuf, [pos + tsplat], g)
            return _
        lax.fori_loop(0, 8, do_tc, 0)
        sds = []
        for tr in range(4):
            sds.append(pltpu.async_copy(
                obuf.at[pl.ds(tr * _TRS, _TRS)],
                out_hbm.at[pl.ds(((f * 4 + tr) * 128 + q * 8) * 1024, _TRS)],
                osem,
            ))
        for sdd in sds:
            sdd.wait()
        return _

    lax.fori_loop(wid * _UNITS_PW, (wid + 1) * _UNITS_PW, do_unit, 0)


def kernel(x, table):
    xt = x.T.astype(jnp.int32)
    idx = _xprep(xt)
    o5 = _gather5(idx, table)
    return (
        o5.reshape(_F, 4, 128, 8, 128)
        .transpose(2, 4, 0, 1, 3)
        .reshape(_B, _F, _D)
    )
